# baseline (speedup 1.0000x reference)
import jax
import jax.numpy as jnp
from jax import lax
from jax.experimental import pallas as pl
from jax.experimental.pallas import tpu as pltpu
from jax.experimental.pallas import tpu_sc as plsc

B = 16384
D = 64
PAIR = 2 * D

_NC = 2
_NS = 16
_NW = _NC * _NS
_BPW = B // _NW
_CHUNK = 64
_NCHUNK = _BPW // _CHUNK


def _sc_gather_body(uid_hbm, iid_hbm, gu_tab, gi_tab, mu_tab, mi_tab,
                    gu_out, gi_out, mu_out, mi_out,
                    idx_mu, idx_su, idx_mi, idx_si, tiles, outs, sems):
    wid = lax.axis_index("s") * _NC + lax.axis_index("c")
    base = wid * _BPW
    pltpu.sync_copy(uid_hbm.at[pl.ds(base, _BPW)], idx_mu)
    pltpu.sync_copy(iid_hbm.at[pl.ds(base, _BPW)], idx_mi)
    for v in range(_BPW // 16):
        sl = pl.ds(v * 16, 16)
        ru = idx_mu[sl]
        idx_su[sl] = lax.bitwise_and(ru, 1)
        idx_mu[sl] = lax.shift_right_logical(ru, 1)
        ri = idx_mi[sl]
        idx_si[sl] = lax.bitwise_and(ri, 1)
        idx_mi[sl] = lax.shift_right_logical(ri, 1)

    tabs = (gu_tab, gi_tab, mu_tab, mi_tab)
    outs_hbm = (gu_out, gi_out, mu_out, mi_out)
    idx_m = (idx_mu, idx_mi, idx_mu, idx_mi)
    idx_s = (idx_su, idx_si, idx_su, idx_si)

    def chunk_body(j, carry):
        row0 = base + j * _CHUNK
        csl = pl.ds(j * _CHUNK, _CHUNK)
        copies = []
        for t in range(4):
            copies.append(pltpu.async_copy(
                tabs[t].at[idx_m[t].at[csl]], tiles.at[t], sems.at[t]))
        for t in range(4):
            copies[t].wait()
            for kk in range(_CHUNK // 16):
                sv = idx_s[t][pl.ds(j * _CHUNK + kk * 16, 16)]
                for l in range(16):
                    k = kk * 16 + l
                    off = sv[l] * D
                    for c in range(D // 16):
                        outs[t, k, pl.ds(c * 16, 16)] = tiles[t, k, pl.ds(off + c * 16, 16)]
            pltpu.sync_copy(outs.at[t], outs_hbm[t].at[pl.ds(row0, _CHUNK)])
        return carry

    lax.fori_loop(0, _NCHUNK, chunk_body, 0, unroll=False)


def _sc_gather(user_ids, item_ids, gu_tab, gi_tab, mu_tab, mi_tab):
    mesh = plsc.VectorSubcoreMesh(core_axis_name="c", subcore_axis_name="s")
    out = jax.ShapeDtypeStruct((B, D), jnp.float32)
    return pl.kernel(
        _sc_gather_body,
        out_type=(out, out, out, out),
        mesh=mesh,
        scratch_types=[
            pltpu.VMEM((_BPW,), jnp.int32),
            pltpu.VMEM((_BPW,), jnp.int32),
            pltpu.VMEM((_BPW,), jnp.int32),
            pltpu.VMEM((_BPW,), jnp.int32),
            pltpu.VMEM((4, _CHUNK, PAIR), jnp.float32),
            pltpu.VMEM((4, _CHUNK, D), jnp.float32),
            pltpu.SemaphoreType.DMA((4,)),
        ],
    )(user_ids, item_ids, gu_tab, gi_tab, mu_tab, mi_tab)


_BK = 2048


def _tc_mlp_body(gu_ref, gi_ref, mu_ref, mi_ref,
                 w1_ref, b1_ref, w2_ref, b2_ref, w3_ref, b3_ref,
                 wo_ref, bo_ref, out_ref):
    f32 = jnp.float32
    gmf = gu_ref[...] * gi_ref[...]
    h = jnp.dot(mu_ref[...], w1_ref[0:D, :], preferred_element_type=f32)
    h += jnp.dot(mi_ref[...], w1_ref[D:2 * D, :], preferred_element_type=f32)
    h = jnp.maximum(h + b1_ref[...], 0.0)
    h = jnp.maximum(jnp.dot(h, w2_ref[...], preferred_element_type=f32) + b2_ref[...], 0.0)
    h = jnp.maximum(jnp.dot(h, w3_ref[...], preferred_element_type=f32) + b3_ref[...], 0.0)
    pred = jnp.dot(gmf, wo_ref[0:D, :], preferred_element_type=f32)
    pred += jnp.dot(h, wo_ref[D:D + 32, :], preferred_element_type=f32)
    out_ref[...] = pred[:, 0] + bo_ref[0]


def _tc_mlp(gu, gi, mu, mi, W1, b1, W2, b2, W3, b3, Wout, bout):
    grid = (B // _BK,)
    row_spec = pl.BlockSpec((_BK, D), lambda i: (i, 0))
    full = lambda shape: pl.BlockSpec(shape, lambda i: tuple(0 for _ in shape))
    return pl.pallas_call(
        _tc_mlp_body,
        grid=grid,
        in_specs=[
            row_spec, row_spec, row_spec, row_spec,
            full(W1.shape), full(b1.shape), full(W2.shape), full(b2.shape),
            full(W3.shape), full(b3.shape), full(Wout.shape), full(bout.shape),
        ],
        out_specs=pl.BlockSpec((_BK,), lambda i: (i,)),
        out_shape=jax.ShapeDtypeStruct((B,), jnp.float32),
    )(gu, gi, mu, mi, W1, b1, W2, b2, W3, b3, Wout, bout)


def kernel(user_ids, item_ids, gmf_user_table, gmf_item_table, mlp_user_table,
           mlp_item_table, W1, b1, W2, b2, W3, b3, Wout, bout):
    uid = user_ids.astype(jnp.int32)
    iid = item_ids.astype(jnp.int32)
    n_pairs = gmf_user_table.shape[0] // 2
    view = lambda t: t.reshape(n_pairs, PAIR)
    gu, gi, mu, mi = _sc_gather(uid, iid, view(gmf_user_table),
                                view(gmf_item_table), view(mlp_user_table),
                                view(mlp_item_table))
    return _tc_mlp(gu, gi, mu, mi, W1, b1, W2, b2, W3, b3, Wout, bout)


# speedup vs baseline: 2.3159x; 2.3159x over previous
import jax
import jax.numpy as jnp
from jax import lax
from jax.experimental import pallas as pl
from jax.experimental.pallas import tpu as pltpu
from jax.experimental.pallas import tpu_sc as plsc

B = 16384
D = 64
N_ROWS = 1000000
PAIR = 2 * D


_TBLK = 4096


def _tc_pack_body(a_ref, b_ref, c_ref, d_ref, oa_ref, ob_ref, oc_ref, od_ref):
    h = _TBLK // 2
    for r, o in ((a_ref, oa_ref), (b_ref, ob_ref), (c_ref, oc_ref), (d_ref, od_ref)):
        xt = jnp.transpose(r[...])
        o[:, 0:D] = xt[0:h, :]
        o[:, D:PAIR] = xt[h:_TBLK, :]


_NGRID = (N_ROWS + _TBLK - 1) // _TBLK
_NPACK = _NGRID * (_TBLK // 2)


def _tc_pack(a, b, c, d):
    grid = (_NGRID,)
    in_spec = pl.BlockSpec((D, _TBLK), lambda i: (0, i))
    out_spec = pl.BlockSpec((_TBLK // 2, PAIR), lambda i: (i, 0))
    out = jax.ShapeDtypeStruct((_NPACK, PAIR), jnp.float32)
    return pl.pallas_call(
        _tc_pack_body,
        grid=grid,
        in_specs=[in_spec] * 4,
        out_specs=[out_spec] * 4,
        out_shape=[out] * 4,
    )(a, b, c, d)


_NC = 2
_NS = 16
_NW = _NC * _NS
_BPW = B // _NW
_CHUNK = 64
_NCHUNK = _BPW // _CHUNK


def _sc_gather_body(uid_hbm, iid_hbm, gu_tab, gi_tab, mu_tab, mi_tab,
                    gu_out, gi_out, mu_out, mi_out,
                    idx_mu, idx_su, idx_mi, idx_si, tiles, outs, sems):
    wid = lax.axis_index("s") * _NC + lax.axis_index("c")
    base = wid * _BPW
    himask = -_TBLK
    lomask = _TBLK // 2 - 1
    sshift = _TBLK.bit_length() - 2
    pltpu.sync_copy(uid_hbm.at[pl.ds(base, _BPW)], idx_mu)
    pltpu.sync_copy(iid_hbm.at[pl.ds(base, _BPW)], idx_mi)
    for v in range(_BPW // 16):
        sl = pl.ds(v * 16, 16)
        ru = idx_mu[sl]
        idx_su[sl] = lax.bitwise_and(lax.shift_right_logical(ru, sshift), 1)
        idx_mu[sl] = lax.bitwise_or(
            lax.shift_right_logical(lax.bitwise_and(ru, himask), 1),
            lax.bitwise_and(ru, lomask))
        ri = idx_mi[sl]
        idx_si[sl] = lax.bitwise_and(lax.shift_right_logical(ri, sshift), 1)
        idx_mi[sl] = lax.bitwise_or(
            lax.shift_right_logical(lax.bitwise_and(ri, himask), 1),
            lax.bitwise_and(ri, lomask))

    tabs = (gu_tab, gi_tab, mu_tab, mi_tab)
    outs_hbm = (gu_out, gi_out, mu_out, mi_out)
    idx_m = (idx_mu, idx_mi, idx_mu, idx_mi)
    idx_s = (idx_su, idx_si, idx_su, idx_si)

    def chunk_body(j, carry):
        row0 = base + j * _CHUNK
        csl = pl.ds(j * _CHUNK, _CHUNK)
        copies = []
        for t in range(4):
            copies.append(pltpu.async_copy(
                tabs[t].at[idx_m[t].at[csl]], tiles.at[t], sems.at[t]))
        for t in range(4):
            copies[t].wait()
            for kk in range(_CHUNK // 16):
                sv = idx_s[t][pl.ds(j * _CHUNK + kk * 16, 16)]
                for l in range(16):
                    k = kk * 16 + l
                    off = sv[l] * D
                    for c in range(D // 16):
                        outs[t, k, pl.ds(c * 16, 16)] = tiles[t, k, pl.ds(off + c * 16, 16)]
            pltpu.sync_copy(outs.at[t], outs_hbm[t].at[pl.ds(row0, _CHUNK)])
        return carry

    lax.fori_loop(0, _NCHUNK, chunk_body, 0, unroll=False)


def _sc_gather(user_ids, item_ids, gu_tab, gi_tab, mu_tab, mi_tab):
    mesh = plsc.VectorSubcoreMesh(core_axis_name="c", subcore_axis_name="s")
    out = jax.ShapeDtypeStruct((B, D), jnp.float32)
    return pl.kernel(
        _sc_gather_body,
        out_type=(out, out, out, out),
        mesh=mesh,
        scratch_types=[
            pltpu.VMEM((_BPW,), jnp.int32),
            pltpu.VMEM((_BPW,), jnp.int32),
            pltpu.VMEM((_BPW,), jnp.int32),
            pltpu.VMEM((_BPW,), jnp.int32),
            pltpu.VMEM((4, _CHUNK, PAIR), jnp.float32),
            pltpu.VMEM((4, _CHUNK, D), jnp.float32),
            pltpu.SemaphoreType.DMA((4,)),
        ],
    )(user_ids, item_ids, gu_tab, gi_tab, mu_tab, mi_tab)


_BK = 2048


def _tc_mlp_body(gu_ref, gi_ref, mu_ref, mi_ref,
                 w1_ref, b1_ref, w2_ref, b2_ref, w3_ref, b3_ref,
                 wo_ref, bo_ref, out_ref):
    f32 = jnp.float32
    gmf = gu_ref[...] * gi_ref[...]
    h = jnp.dot(mu_ref[...], w1_ref[0:D, :], preferred_element_type=f32)
    h += jnp.dot(mi_ref[...], w1_ref[D:2 * D, :], preferred_element_type=f32)
    h = jnp.maximum(h + b1_ref[...], 0.0)
    h = jnp.maximum(jnp.dot(h, w2_ref[...], preferred_element_type=f32) + b2_ref[...], 0.0)
    h = jnp.maximum(jnp.dot(h, w3_ref[...], preferred_element_type=f32) + b3_ref[...], 0.0)
    pred = jnp.dot(gmf, wo_ref[0:D, :], preferred_element_type=f32)
    pred += jnp.dot(h, wo_ref[D:D + 32, :], preferred_element_type=f32)
    out_ref[...] = pred[:, 0] + bo_ref[0]


def _tc_mlp(gu, gi, mu, mi, W1, b1, W2, b2, W3, b3, Wout, bout):
    grid = (B // _BK,)
    row_spec = pl.BlockSpec((_BK, D), lambda i: (i, 0))
    full = lambda shape: pl.BlockSpec(shape, lambda i: tuple(0 for _ in shape))
    return pl.pallas_call(
        _tc_mlp_body,
        grid=grid,
        in_specs=[
            row_spec, row_spec, row_spec, row_spec,
            full(W1.shape), full(b1.shape), full(W2.shape), full(b2.shape),
            full(W3.shape), full(b3.shape), full(Wout.shape), full(bout.shape),
        ],
        out_specs=pl.BlockSpec((_BK,), lambda i: (i,)),
        out_shape=jax.ShapeDtypeStruct((B,), jnp.float32),
    )(gu, gi, mu, mi, W1, b1, W2, b2, W3, b3, Wout, bout)


def kernel(user_ids, item_ids, gmf_user_table, gmf_item_table, mlp_user_table,
           mlp_item_table, W1, b1, W2, b2, W3, b3, Wout, bout):
    uid = user_ids.astype(jnp.int32)
    iid = item_ids.astype(jnp.int32)
    pgu, pgi, pmu, pmi = _tc_pack(gmf_user_table.T, gmf_item_table.T,
                                  mlp_user_table.T, mlp_item_table.T)
    gu, gi, mu, mi = _sc_gather(uid, iid, pgu, pgi, pmu, pmi)
    return _tc_mlp(gu, gi, mu, mi, W1, b1, W2, b2, W3, b3, Wout, bout)


# speedup vs baseline: 3.6850x; 1.5912x over previous
import jax
import jax.numpy as jnp
from jax import lax
from jax.experimental import pallas as pl
from jax.experimental.pallas import tpu as pltpu
from jax.experimental.pallas import tpu_sc as plsc

B = 16384
D = 64
N_ROWS = 1000000
PAIR = 2 * D


_TBLK = 4096


def _tc_pack_body(a_ref, b_ref, c_ref, d_ref, oa_ref, ob_ref, oc_ref, od_ref):
    q = _TBLK // 4
    for r, o in ((a_ref, oa_ref), (b_ref, ob_ref), (c_ref, oc_ref), (d_ref, od_ref)):
        x = r[...]
        lo = lax.bitcast_convert_type(x[0:D // 2, :], jnp.int32)
        hi = lax.bitcast_convert_type(x[D // 2:D, :], jnp.int32)
        lo = lax.shift_right_logical(lo + 0x8000, 16)
        hi = lax.bitwise_and(hi + 0x8000, -65536)
        y = lax.bitwise_or(lo, hi)
        z = jnp.concatenate([y[:, s * q:(s + 1) * q] for s in range(4)], axis=0)
        o[...] = jnp.transpose(z)


_NGRID = (N_ROWS + _TBLK - 1) // _TBLK
_NPACK = _NGRID * (_TBLK // 4)


def _tc_pack(a, b, c, d):
    grid = (_NGRID,)
    in_spec = pl.BlockSpec((D, _TBLK), lambda i: (0, i))
    out_spec = pl.BlockSpec((_TBLK // 4, PAIR), lambda i: (i, 0))
    out = jax.ShapeDtypeStruct((_NPACK, PAIR), jnp.int32)
    return pl.pallas_call(
        _tc_pack_body,
        grid=grid,
        in_specs=[in_spec] * 4,
        out_specs=[out_spec] * 4,
        out_shape=[out] * 4,
    )(a, b, c, d)


_NC = 2
_NS = 16
_NW = _NC * _NS
_BPW = B // _NW
_CHUNK = 64
_NCHUNK = _BPW // _CHUNK


def _sc_gather_body(uid_hbm, iid_hbm, gu_tab, gi_tab, mu_tab, mi_tab,
                    gu_out, gi_out, mu_out, mi_out,
                    idx_mu, idx_su, idx_mi, idx_si, tiles, outs, sems):
    wid = lax.axis_index("s") * _NC + lax.axis_index("c")
    base = wid * _BPW
    himask = -_TBLK
    lomask = _TBLK // 4 - 1
    sshift = _TBLK.bit_length() - 3
    pltpu.sync_copy(uid_hbm.at[pl.ds(base, _BPW)], idx_mu)
    pltpu.sync_copy(iid_hbm.at[pl.ds(base, _BPW)], idx_mi)
    for v in range(_BPW // 16):
        sl = pl.ds(v * 16, 16)
        ru = idx_mu[sl]
        idx_su[sl] = lax.bitwise_and(lax.shift_right_logical(ru, sshift), 3)
        idx_mu[sl] = lax.bitwise_or(
            lax.shift_right_logical(lax.bitwise_and(ru, himask), 2),
            lax.bitwise_and(ru, lomask))
        ri = idx_mi[sl]
        idx_si[sl] = lax.bitwise_and(lax.shift_right_logical(ri, sshift), 3)
        idx_mi[sl] = lax.bitwise_or(
            lax.shift_right_logical(lax.bitwise_and(ri, himask), 2),
            lax.bitwise_and(ri, lomask))

    tabs = (gu_tab, gi_tab, mu_tab, mi_tab)
    outs_hbm = (gu_out, gi_out, mu_out, mi_out)
    idx_m = (idx_mu, idx_mi, idx_mu, idx_mi)
    idx_s = (idx_su, idx_si, idx_su, idx_si)

    def chunk_body(j, carry):
        row0 = base + j * _CHUNK
        csl = pl.ds(j * _CHUNK, _CHUNK)
        copies = []
        for t in range(4):
            copies.append(pltpu.async_copy(
                tabs[t].at[idx_m[t].at[csl]], tiles.at[t], sems.at[t]))
        for t in range(4):
            copies[t].wait()
            for kk in range(_CHUNK // 16):
                sv = idx_s[t][pl.ds(j * _CHUNK + kk * 16, 16)]
                for l in range(16):
                    k = kk * 16 + l
                    off = sv[l] * 32
                    for c in range(2):
                        outs[t, k, pl.ds(c * 16, 16)] = tiles[t, k, pl.ds(off + c * 16, 16)]
            pltpu.sync_copy(outs.at[t], outs_hbm[t].at[pl.ds(row0, _CHUNK)])
        return carry

    lax.fori_loop(0, _NCHUNK, chunk_body, 0, unroll=False)


def _sc_gather(user_ids, item_ids, gu_tab, gi_tab, mu_tab, mi_tab):
    mesh = plsc.VectorSubcoreMesh(core_axis_name="c", subcore_axis_name="s")
    out = jax.ShapeDtypeStruct((B, D // 2), jnp.int32)
    return pl.kernel(
        _sc_gather_body,
        out_type=(out, out, out, out),
        mesh=mesh,
        scratch_types=[
            pltpu.VMEM((_BPW,), jnp.int32),
            pltpu.VMEM((_BPW,), jnp.int32),
            pltpu.VMEM((_BPW,), jnp.int32),
            pltpu.VMEM((_BPW,), jnp.int32),
            pltpu.VMEM((4, _CHUNK, PAIR), jnp.int32),
            pltpu.VMEM((4, _CHUNK, D // 2), jnp.int32),
            pltpu.SemaphoreType.DMA((4,)),
        ],
    )(user_ids, item_ids, gu_tab, gi_tab, mu_tab, mi_tab)


_BK = 2048


def _tc_mlp_body(gu_ref, gi_ref, mu_ref, mi_ref,
                 w1_ref, b1_ref, w2_ref, b2_ref, w3_ref, b3_ref,
                 wo_ref, bo_ref, out_ref):
    f32 = jnp.float32
    H = D // 2

    def unpack(ref):
        x = ref[...]
        lo = lax.bitcast_convert_type(lax.shift_left(x, 16), f32)
        hi = lax.bitcast_convert_type(lax.bitwise_and(x, -65536), f32)
        return lo, hi

    gul, guh = unpack(gu_ref)
    gil, gih = unpack(gi_ref)
    mul, muh = unpack(mu_ref)
    mil, mih = unpack(mi_ref)
    h = jnp.dot(mul, w1_ref[0:H, :], preferred_element_type=f32)
    h += jnp.dot(muh, w1_ref[H:2 * H, :], preferred_element_type=f32)
    h += jnp.dot(mil, w1_ref[2 * H:3 * H, :], preferred_element_type=f32)
    h += jnp.dot(mih, w1_ref[3 * H:4 * H, :], preferred_element_type=f32)
    h = jnp.maximum(h + b1_ref[...], 0.0)
    h = jnp.maximum(jnp.dot(h, w2_ref[...], preferred_element_type=f32) + b2_ref[...], 0.0)
    h = jnp.maximum(jnp.dot(h, w3_ref[...], preferred_element_type=f32) + b3_ref[...], 0.0)
    pred = jnp.dot(gul * gil, wo_ref[0:H, :], preferred_element_type=f32)
    pred += jnp.dot(guh * gih, wo_ref[H:2 * H, :], preferred_element_type=f32)
    pred += jnp.dot(h, wo_ref[D:D + 32, :], preferred_element_type=f32)
    out_ref[...] = pred[:, 0] + bo_ref[0]


def _tc_mlp(gu, gi, mu, mi, W1, b1, W2, b2, W3, b3, Wout, bout):
    grid = (B // _BK,)
    row_spec = pl.BlockSpec((_BK, D // 2), lambda i: (i, 0))
    full = lambda shape: pl.BlockSpec(shape, lambda i: tuple(0 for _ in shape))
    return pl.pallas_call(
        _tc_mlp_body,
        grid=grid,
        in_specs=[
            row_spec, row_spec, row_spec, row_spec,
            full(W1.shape), full(b1.shape), full(W2.shape), full(b2.shape),
            full(W3.shape), full(b3.shape), full(Wout.shape), full(bout.shape),
        ],
        out_specs=pl.BlockSpec((_BK,), lambda i: (i,)),
        out_shape=jax.ShapeDtypeStruct((B,), jnp.float32),
    )(gu, gi, mu, mi, W1, b1, W2, b2, W3, b3, Wout, bout)


def kernel(user_ids, item_ids, gmf_user_table, gmf_item_table, mlp_user_table,
           mlp_item_table, W1, b1, W2, b2, W3, b3, Wout, bout):
    uid = user_ids.astype(jnp.int32)
    iid = item_ids.astype(jnp.int32)
    pgu, pgi, pmu, pmi = _tc_pack(gmf_user_table.T, gmf_item_table.T,
                                  mlp_user_table.T, mlp_item_table.T)
    gu, gi, mu, mi = _sc_gather(uid, iid, pgu, pgi, pmu, pmi)
    return _tc_mlp(gu, gi, mu, mi, W1, b1, W2, b2, W3, b3, Wout, bout)


# speedup vs baseline: 3.6997x; 1.0040x over previous
import jax
import jax.numpy as jnp
from jax import lax
from jax.experimental import pallas as pl
from jax.experimental.pallas import tpu as pltpu
from jax.experimental.pallas import tpu_sc as plsc

B = 16384
D = 64
N_ROWS = 1000000
PAIR = 2 * D


_TBLK = 8192
_QSHIFT = _TBLK.bit_length() - 3


def _tc_pack_body(a_ref, b_ref, c_ref, d_ref, oa_ref, ob_ref, oc_ref, od_ref):
    q = _TBLK // 4
    for r, o in ((a_ref, oa_ref), (b_ref, ob_ref), (c_ref, oc_ref), (d_ref, od_ref)):
        x = r[...]
        lo = lax.bitcast_convert_type(x[0:D // 2, :], jnp.int32)
        hi = lax.bitcast_convert_type(x[D // 2:D, :], jnp.int32)
        lo = lax.shift_right_logical(lo + 0x8000, 16)
        hi = lax.bitwise_and(hi + 0x8000, -65536)
        y = lax.bitwise_or(lo, hi)
        z = jnp.concatenate([y[:, s * q:(s + 1) * q] for s in range(4)], axis=0)
        o[...] = jnp.transpose(z)


_NGRID = (N_ROWS + _TBLK - 1) // _TBLK
_NPACK = _NGRID * (_TBLK // 4)


def _tc_pack(a, b, c, d):
    grid = (_NGRID,)
    in_spec = pl.BlockSpec((D, _TBLK), lambda i: (0, i))
    out_spec = pl.BlockSpec((_TBLK // 4, PAIR), lambda i: (i, 0))
    out = jax.ShapeDtypeStruct((_NPACK, PAIR), jnp.int32)
    return pl.pallas_call(
        _tc_pack_body,
        grid=grid,
        in_specs=[in_spec] * 4,
        out_specs=[out_spec] * 4,
        out_shape=[out] * 4,
    )(a, b, c, d)


_NC = 2
_NS = 16
_NW = _NC * _NS
_BPW = B // _NW
_CHUNK = 64
_NCHUNK = _BPW // _CHUNK


def _sc_gather_body(uid_hbm, iid_hbm, gu_tab, gi_tab, mu_tab, mi_tab,
                    gu_out, gi_out, mu_out, mi_out,
                    idx_mu, idx_mi, tiles, sems):
    wid = lax.axis_index("s") * _NC + lax.axis_index("c")
    base = wid * _BPW
    himask = -_TBLK
    lomask = _TBLK // 4 - 1
    pltpu.sync_copy(uid_hbm.at[pl.ds(base, _BPW)], idx_mu)
    pltpu.sync_copy(iid_hbm.at[pl.ds(base, _BPW)], idx_mi)
    for v in range(_BPW // 16):
        sl = pl.ds(v * 16, 16)
        ru = idx_mu[sl]
        idx_mu[sl] = lax.bitwise_or(
            lax.shift_right_logical(lax.bitwise_and(ru, himask), 2),
            lax.bitwise_and(ru, lomask))
        ri = idx_mi[sl]
        idx_mi[sl] = lax.bitwise_or(
            lax.shift_right_logical(lax.bitwise_and(ri, himask), 2),
            lax.bitwise_and(ri, lomask))

    tabs = (gu_tab, gi_tab, mu_tab, mi_tab)
    outs_hbm = (gu_out, gi_out, mu_out, mi_out)
    idx_m = (idx_mu, idx_mi, idx_mu, idx_mi)

    def chunk_body(j, carry):
        row0 = base + j * _CHUNK
        csl = pl.ds(j * _CHUNK, _CHUNK)
        copies = []
        for t in range(4):
            copies.append(pltpu.async_copy(
                tabs[t].at[idx_m[t].at[csl]], tiles.at[t], sems.at[t]))
        for t in range(4):
            copies[t].wait()
            pltpu.sync_copy(tiles.at[t], outs_hbm[t].at[pl.ds(row0, _CHUNK)])
        return carry

    lax.fori_loop(0, _NCHUNK, chunk_body, 0, unroll=False)


def _sc_gather(user_ids, item_ids, gu_tab, gi_tab, mu_tab, mi_tab):
    mesh = plsc.VectorSubcoreMesh(core_axis_name="c", subcore_axis_name="s")
    out = jax.ShapeDtypeStruct((B, PAIR), jnp.int32)
    return pl.kernel(
        _sc_gather_body,
        out_type=(out, out, out, out),
        mesh=mesh,
        scratch_types=[
            pltpu.VMEM((_BPW,), jnp.int32),
            pltpu.VMEM((_BPW,), jnp.int32),
            pltpu.VMEM((4, _CHUNK, PAIR), jnp.int32),
            pltpu.SemaphoreType.DMA((4,)),
        ],
    )(user_ids, item_ids, gu_tab, gi_tab, mu_tab, mi_tab)


_BK = 4096


def _tc_mlp_body(uid_ref, iid_ref, gu_ref, gi_ref, mu_ref, mi_ref,
                 w1_ref, b1_ref, w2_ref, b2_ref, w3_ref, b3_ref,
                 wo_ref, bo_ref, out_ref):
    f32 = jnp.float32
    H = D // 2
    us2 = lax.bitwise_and(lax.shift_right_logical(uid_ref[...], _QSHIFT), 3)
    is2 = lax.bitwise_and(lax.shift_right_logical(iid_ref[...], _QSHIFT), 3)

    def unpack(ref, s2):
        lines = ref[...]
        x = lines[:, 0:H]
        for s in range(1, 4):
            x = jnp.where(s2 == s, lines[:, s * H:(s + 1) * H], x)
        lo = lax.bitcast_convert_type(lax.shift_left(x, 16), f32)
        hi = lax.bitcast_convert_type(lax.bitwise_and(x, -65536), f32)
        return lo, hi

    gul, guh = unpack(gu_ref, us2)
    gil, gih = unpack(gi_ref, is2)
    mul, muh = unpack(mu_ref, us2)
    mil, mih = unpack(mi_ref, is2)
    h = jnp.dot(mul, w1_ref[0:H, :], preferred_element_type=f32)
    h += jnp.dot(muh, w1_ref[H:2 * H, :], preferred_element_type=f32)
    h += jnp.dot(mil, w1_ref[2 * H:3 * H, :], preferred_element_type=f32)
    h += jnp.dot(mih, w1_ref[3 * H:4 * H, :], preferred_element_type=f32)
    h = jnp.maximum(h + b1_ref[...], 0.0)
    h = jnp.maximum(jnp.dot(h, w2_ref[...], preferred_element_type=f32) + b2_ref[...], 0.0)
    h = jnp.maximum(jnp.dot(h, w3_ref[...], preferred_element_type=f32) + b3_ref[...], 0.0)
    pred = jnp.dot(gul * gil, wo_ref[0:H, :], preferred_element_type=f32)
    pred += jnp.dot(guh * gih, wo_ref[H:2 * H, :], preferred_element_type=f32)
    pred += jnp.dot(h, wo_ref[D:D + 32, :], preferred_element_type=f32)
    out_ref[...] = pred[:, 0] + bo_ref[0]


def _tc_mlp(uid, iid, gu, gi, mu, mi, W1, b1, W2, b2, W3, b3, Wout, bout):
    grid = (B // _BK,)
    id_spec = pl.BlockSpec((_BK, 1), lambda i: (i, 0))
    row_spec = pl.BlockSpec((_BK, PAIR), lambda i: (i, 0))
    full = lambda shape: pl.BlockSpec(shape, lambda i: tuple(0 for _ in shape))
    return pl.pallas_call(
        _tc_mlp_body,
        grid=grid,
        in_specs=[
            id_spec, id_spec, row_spec, row_spec, row_spec, row_spec,
            full(W1.shape), full(b1.shape), full(W2.shape), full(b2.shape),
            full(W3.shape), full(b3.shape), full(Wout.shape), full(bout.shape),
        ],
        out_specs=pl.BlockSpec((_BK,), lambda i: (i,)),
        out_shape=jax.ShapeDtypeStruct((B,), jnp.float32),
    )(uid.reshape(B, 1), iid.reshape(B, 1), gu, gi, mu, mi,
      W1, b1, W2, b2, W3, b3, Wout, bout)


def kernel(user_ids, item_ids, gmf_user_table, gmf_item_table, mlp_user_table,
           mlp_item_table, W1, b1, W2, b2, W3, b3, Wout, bout):
    uid = user_ids.astype(jnp.int32)
    iid = item_ids.astype(jnp.int32)
    pgu, pgi, pmu, pmi = _tc_pack(gmf_user_table.T, gmf_item_table.T,
                                  mlp_user_table.T, mlp_item_table.T)
    gu, gi, mu, mi = _sc_gather(uid, iid, pgu, pgi, pmu, pmi)
    return _tc_mlp(uid, iid, gu, gi, mu, mi, W1, b1, W2, b2, W3, b3, Wout, bout)
